# baseline (speedup 1.0000x reference)
import functools

import jax
import jax.numpy as jnp
from jax import lax
from jax.experimental import pallas as pl
from jax.experimental.pallas import tpu as pltpu
from jax.experimental.pallas import tpu_sc as plsc

_N_POSES = 100000
_B = 16384

_info = plsc.get_sparse_core_info()
_NC = _info.num_cores
_NS = _info.num_subcores
_NW = _NC * _NS
_BPW = _B // _NW

_mesh = plsc.VectorSubcoreMesh(core_axis_name="c", subcore_axis_name="s")


@functools.partial(
    pl.kernel,
    mesh=_mesh,
    compiler_params=pltpu.CompilerParams(use_tc_tiling_on_sc=False),
    out_type=(
        jax.ShapeDtypeStruct((_B, 4), jnp.float32),
        jax.ShapeDtypeStruct((_B, 3), jnp.float32),
    ),
    scratch_types=[
        pltpu.VMEM((_BPW,), jnp.int32),
        pltpu.VMEM((_BPW, 8), jnp.float32),
        pltpu.SemaphoreType.DMA,
    ],
)
def _gather_poses(idx_hbm, tab_hbm, q_out, t_out, idx_v, row_v, sem):
    wid = lax.axis_index("s") * _NC + lax.axis_index("c")
    base = wid * _BPW
    pltpu.sync_copy(idx_hbm.at[pl.ds(base, _BPW)], idx_v)
    pltpu.async_copy(tab_hbm.at[idx_v], row_v, sem).wait()
    pltpu.sync_copy(row_v.at[:, pl.ds(0, 4)], q_out.at[pl.ds(base, _BPW)])
    pltpu.sync_copy(row_v.at[:, pl.ds(4, 3)], t_out.at[pl.ds(base, _BPW)])


def kernel(camera_pose_indices, q_camera_pointcloud_table,
           t_camera_pointcloud_table):
    idx = camera_pose_indices.astype(jnp.int32)
    n = q_camera_pointcloud_table.shape[0]
    fused = jnp.concatenate(
        [q_camera_pointcloud_table,
         t_camera_pointcloud_table,
         jnp.zeros((n, 1), jnp.float32)], axis=1)
    return _gather_poses(idx, fused)


# speedup vs baseline: 1.5101x; 1.5101x over previous
import functools

import jax
import jax.numpy as jnp
from jax import lax
from jax.experimental import pallas as pl
from jax.experimental.pallas import tpu as pltpu
from jax.experimental.pallas import tpu_sc as plsc

_N = 100000
_B = 16384

_info = plsc.get_sparse_core_info()
_NC = _info.num_cores
_NS = _info.num_subcores
_NW = _NC * _NS
_BPW = _B // _NW

_mesh = plsc.VectorSubcoreMesh(core_axis_name="c", subcore_axis_name="s")


@functools.partial(
    pl.kernel,
    mesh=_mesh,
    compiler_params=pltpu.CompilerParams(
        use_tc_tiling_on_sc=False, needs_layout_passes=False),
    out_type=(
        jax.ShapeDtypeStruct((_B * 4,), jnp.float32),
        jax.ShapeDtypeStruct((_B * 3,), jnp.float32),
    ),
    scratch_types=[
        pltpu.VMEM((_BPW,), jnp.int32),
        pltpu.VMEM((_BPW, 8), jnp.float32),
        pltpu.VMEM((_BPW * 4,), jnp.float32),
        pltpu.VMEM((_BPW * 3,), jnp.float32),
        pltpu.SemaphoreType.DMA,
    ],
)
def _gather_poses(idx_hbm, tab_hbm, q_out, t_out,
                  idx_v, row_v, q_v, t_v, sem):
    wid = lax.axis_index("s") * _NC + lax.axis_index("c")
    base = wid * _BPW
    pltpu.sync_copy(idx_hbm.at[pl.ds(base, _BPW)], idx_v)
    pltpu.async_copy(tab_hbm.at[idx_v], row_v, sem).wait()
    lane = lax.iota(jnp.int32, 16)

    def qbody(c, _):
        m = 16 * c + lane
        q_v[pl.ds(16 * c, 16)] = plsc.load_gather(row_v, [m >> 2, m & 3])
        return 0

    def tbody(c, _):
        m = 16 * c + lane
        r = (m * 21846) >> 16
        t_v[pl.ds(16 * c, 16)] = plsc.load_gather(row_v, [r, m - 3 * r + 4])
        return 0

    lax.fori_loop(0, _BPW * 4 // 16, qbody, 0)
    lax.fori_loop(0, _BPW * 3 // 16, tbody, 0)
    pltpu.sync_copy(q_v, q_out.at[pl.ds(base * 4, _BPW * 4)])
    pltpu.sync_copy(t_v, t_out.at[pl.ds(base * 3, _BPW * 3)])


def kernel(camera_pose_indices, q_camera_pointcloud_table,
           t_camera_pointcloud_table):
    idx = camera_pose_indices.astype(jnp.int32)
    n = q_camera_pointcloud_table.shape[0]
    fused = jnp.concatenate(
        [q_camera_pointcloud_table,
         t_camera_pointcloud_table,
         jnp.zeros((n, 1), jnp.float32)], axis=1)
    qf, tf = _gather_poses(idx, fused)
    return qf.reshape(_B, 4), tf.reshape(_B, 3)
